# baseline (speedup 1.0000x reference)
import functools

import jax
import jax.numpy as jnp
from jax import lax
from jax.experimental import pallas as pl
from jax.experimental.pallas import tpu as pltpu
from jax.experimental.pallas import tpu_sc as plsc

EMBED = 128
N_EDGES = 320000
NC = 2
NS = 16
NW = NC * NS
PER_W = N_EDGES // NW
CH = 80
NCHUNK = PER_W // CH


def _tec_body(table_hbm, idx_hbm, out_hbm, idx_v, rows_v, gsem):
    wid = lax.axis_index("s") * NC + lax.axis_index("c")
    pltpu.sync_copy(idx_hbm.at[wid], idx_v)
    base = wid * PER_W

    def chunk(g, carry):
        pltpu.async_copy(table_hbm.at[idx_v.at[g]], rows_v, gsem).wait()
        pltpu.sync_copy(rows_v, out_hbm.at[pl.ds(base + g * CH, CH)])
        return carry

    lax.fori_loop(0, NCHUNK, chunk, 0)


_mesh = plsc.VectorSubcoreMesh(core_axis_name="c", subcore_axis_name="s")

_sc_call = pl.kernel(
    _tec_body,
    mesh=_mesh,
    out_type=jax.ShapeDtypeStruct((N_EDGES, EMBED), jnp.float32),
    scratch_types=[
        pltpu.VMEM((NCHUNK, CH), jnp.int32),
        pltpu.VMEM((CH, EMBED), jnp.float32),
        pltpu.SemaphoreType.DMA,
    ],
)


@jax.jit
def _run(data, table):
    idx = data.astype(jnp.int32).reshape(NW, NCHUNK, CH)
    return _sc_call(table, idx)


def kernel(data, edge_type_embedding):
    return _run(data, edge_type_embedding)


# speedup vs baseline: 1.0130x; 1.0130x over previous
import jax
import jax.numpy as jnp
from jax import lax
from jax.experimental import pallas as pl
from jax.experimental.pallas import tpu as pltpu
from jax.experimental.pallas import tpu_sc as plsc

EMBED = 128
N_EDGES = 320000
NC = 2
NS = 16
NW = NC * NS
PER_W = N_EDGES // NW
CH = 80
NCHUNK = PER_W // CH
NBUF = 5
K = 2


def _tec_body(table_hbm, idx_hbm, out_hbm, idx_v, rows_v, gsem, wsem):
    wid = lax.axis_index("s") * NC + lax.axis_index("c")
    pltpu.sync_copy(idx_hbm.at[wid], idx_v)
    base = wid * PER_W

    def gather_wait(b):
        pltpu.make_async_copy(
            out_hbm.at[pl.ds(0, CH)], rows_v.at[b], gsem.at[b]).wait()

    def write_wait(b):
        pltpu.make_async_copy(
            rows_v.at[b], out_hbm.at[pl.ds(base, CH)], wsem.at[b]).wait()

    for c in range(K):
        pltpu.async_copy(table_hbm.at[idx_v.at[c]], rows_v.at[c], gsem.at[c])

    def outer(t, carry):
        for j in range(NBUF):
            c = t * NBUF + j
            pre = c + K
            bp = (j + K) % NBUF

            @pl.when(pre < NCHUNK)
            def _():
                @pl.when(pre >= NBUF)
                def _():
                    write_wait(bp)
                pltpu.async_copy(
                    table_hbm.at[idx_v.at[pre]], rows_v.at[bp], gsem.at[bp])

            gather_wait(j)
            pltpu.async_copy(
                rows_v.at[j], out_hbm.at[pl.ds(base + c * CH, CH)], wsem.at[j])
        return carry

    lax.fori_loop(0, NCHUNK // NBUF, outer, 0)
    for b in range(NBUF):
        write_wait(b)


_mesh = plsc.VectorSubcoreMesh(core_axis_name="c", subcore_axis_name="s")

_sc_call = pl.kernel(
    _tec_body,
    mesh=_mesh,
    out_type=jax.ShapeDtypeStruct((N_EDGES, EMBED), jnp.float32),
    scratch_types=[
        pltpu.VMEM((NCHUNK, CH), jnp.int32),
        pltpu.VMEM((NBUF, CH, EMBED), jnp.float32),
        pltpu.SemaphoreType.DMA((NBUF,)),
        pltpu.SemaphoreType.DMA((NBUF,)),
    ],
)


@jax.jit
def _run(data, table):
    idx = data.astype(jnp.int32).reshape(NW, NCHUNK, CH)
    return _sc_call(table, idx)


def kernel(data, edge_type_embedding):
    return _run(data, edge_type_embedding)


# speedup vs baseline: 1.8305x; 1.8070x over previous
import jax
import jax.numpy as jnp
from jax import lax
from jax.experimental import pallas as pl
from jax.experimental.pallas import tpu as pltpu
from jax.experimental.pallas import tpu_sc as plsc

EMBED = 128
N_EDGES = 320000
NROWS = 16
NC = 2
NS = 16
NW = NC * NS
N_PAIR = N_EDGES // 2
PER_W = N_PAIR // NW
CH = 40
NCHUNK = PER_W // CH
NBUF = 5
K = 2


def _tec_body(ptab_hbm, idx_hbm, out_hbm, idx_v, rows_v, gsem, wsem):
    wid = lax.axis_index("s") * NC + lax.axis_index("c")
    pltpu.sync_copy(idx_hbm.at[wid], idx_v)
    base = wid * PER_W

    def gather_wait(b):
        pltpu.make_async_copy(
            out_hbm.at[pl.ds(0, CH)], rows_v.at[b], gsem.at[b]).wait()

    def write_wait(b):
        pltpu.make_async_copy(
            rows_v.at[b], out_hbm.at[pl.ds(base, CH)], wsem.at[b]).wait()

    for c in range(K):
        pltpu.async_copy(ptab_hbm.at[idx_v.at[c]], rows_v.at[c], gsem.at[c])

    def outer(t, carry):
        for j in range(NBUF):
            c = t * NBUF + j
            pre = c + K
            bp = (j + K) % NBUF

            @pl.when(pre < NCHUNK)
            def _():
                @pl.when(pre >= NBUF)
                def _():
                    write_wait(bp)
                pltpu.async_copy(
                    ptab_hbm.at[idx_v.at[pre]], rows_v.at[bp], gsem.at[bp])

            gather_wait(j)
            pltpu.async_copy(
                rows_v.at[j], out_hbm.at[pl.ds(base + c * CH, CH)], wsem.at[j])
        return carry

    lax.fori_loop(0, NCHUNK // NBUF, outer, 0)
    for b in range(NBUF):
        write_wait(b)


_mesh = plsc.VectorSubcoreMesh(core_axis_name="c", subcore_axis_name="s")

_sc_call = pl.kernel(
    _tec_body,
    mesh=_mesh,
    out_type=jax.ShapeDtypeStruct((N_PAIR, 2 * EMBED), jnp.float32),
    scratch_types=[
        pltpu.VMEM((NCHUNK, CH), jnp.int32),
        pltpu.VMEM((NBUF, CH, 2 * EMBED), jnp.float32),
        pltpu.SemaphoreType.DMA((NBUF,)),
        pltpu.SemaphoreType.DMA((NBUF,)),
    ],
)


@jax.jit
def _run(data, table):
    ptab = jnp.concatenate(
        [jnp.repeat(table, NROWS, axis=0), jnp.tile(table, (NROWS, 1))],
        axis=1)
    d = data.astype(jnp.int32).reshape(-1, 2)
    pidx = (d[:, 0] * NROWS + d[:, 1]).reshape(NW, NCHUNK, CH)
    out = _sc_call(ptab, pidx)
    return out.reshape(N_EDGES, EMBED)


def kernel(data, edge_type_embedding):
    return _run(data, edge_type_embedding)
